# baseline (speedup 1.0000x reference)
import functools

import jax
import jax.numpy as jnp
from jax import lax
from jax.experimental import pallas as pl
from jax.experimental.pallas import tpu as pltpu
from jax.experimental.pallas import tpu_sc as plsc

B = 16384
F = 26
D = 16
V = 1000000
NC, NS, L = 2, 16, 16
NW = NC * NS
RPW = B // NW
IPW = RPW * F
XW = 128
XROWS_W = IPW // XW
C = 64
NCHUNK = RPW // C
GJ = C * F // XW
GROUPS = C // L

TCF = V // XW
TPW = TCF // NW
TEXTRA = TCF - TPW * NW
TPART = V - TCF * XW

_MESH = plsc.VectorSubcoreMesh(core_axis_name="c", subcore_axis_name="s")
_PARAMS_LIN = pltpu.CompilerParams(needs_layout_passes=False,
                                   use_tc_tiling_on_sc=False)
_PARAMS_TC = pltpu.CompilerParams(needs_layout_passes=False,
                                  use_tc_tiling_on_sc=True)


def _detile_body(embt_hbm, tail_hbm, lin_hbm,
                 in00, in01, in10, in11, out0, out1,
                 rsem0, rsem1, wsem0, wsem1):
    wid = lax.axis_index("s") * NC + lax.axis_index("c")
    base = wid * TPW
    lane = lax.iota(jnp.int32, L)
    l8 = lane % 8
    low = lane < 8
    ins = ((in00, in01), (in10, in11))
    outs = (out0, out1)
    rsems = (rsem0, rsem1)
    wsems = (wsem0, wsem1)

    def read(tc, slot):
        col = pl.multiple_of(tc * XW, XW)
        pltpu.async_copy(embt_hbm.at[pl.ds(0, 8), pl.ds(col, XW)],
                         ins[0][slot], rsems[slot])
        pltpu.async_copy(embt_hbm.at[pl.ds(8, 8), pl.ds(col, XW)],
                         ins[1][slot], rsems[slot])

    def drain_read(slot):
        for h in range(2):
            pltpu.make_async_copy(
                embt_hbm.at[pl.ds(0, 8), pl.ds(0, XW)],
                ins[h][slot], rsems[slot]).wait()

    def drain_write(slot):
        pltpu.make_async_copy(
            outs[slot], lin_hbm.at[pl.ds(0, D * XW)], wsems[slot]).wait()

    def transpose(slot, width):
        i0b, i1b = ins[0][slot], ins[1][slot]
        ob = outs[slot]

        def tbody(k, carry):
            ks = lane * 0 + k
            g0 = plsc.load_gather(i0b, [l8, ks])
            g1 = plsc.load_gather(i1b, [l8, ks])
            ob[pl.ds(k * D, D)] = jnp.where(low, g0, g1)
            return carry
        lax.fori_loop(0, width, tbody, 0)

    read(base, 0)
    read(base + 1, 1)

    def body(it, carry):
        for b in range(2):
            blk = it * 2 + b
            drain_read(b)

            @pl.when(blk >= 2)
            def _():
                drain_write(b)

            transpose(b, XW)
            off = pl.multiple_of((base + blk) * (XW * D), XW * D)
            pltpu.async_copy(outs[b], lin_hbm.at[pl.ds(off, XW * D)],
                             wsems[b])

            @pl.when(blk + 2 < TPW)
            def _():
                read(base + blk + 2, b)
        return carry
    lax.fori_loop(0, TPW // 2, body, 0)
    drain_write(0)
    drain_write(1)

    @pl.when(wid < TEXTRA)
    def _():
        tc = TCF - TEXTRA + wid
        col = pl.multiple_of(tc * XW, XW)
        pltpu.sync_copy(embt_hbm.at[pl.ds(0, 8), pl.ds(col, XW)], in00)
        pltpu.sync_copy(embt_hbm.at[pl.ds(8, 8), pl.ds(col, XW)], in10)
        transpose(0, XW)
        off = pl.multiple_of(tc * (XW * D), XW * D)
        pltpu.sync_copy(out0, lin_hbm.at[pl.ds(off, XW * D)])

    @pl.when(wid == TEXTRA)
    def _():
        pltpu.sync_copy(tail_hbm.at[pl.ds(0, 8), pl.ds(0, XW)], in00)
        pltpu.sync_copy(tail_hbm.at[pl.ds(8, 8), pl.ds(0, XW)], in10)
        transpose(0, TPART)
        pltpu.sync_copy(out0.at[pl.ds(0, TPART * D)],
                        lin_hbm.at[pl.ds(TCF * XW * D, TPART * D)])


_detile = functools.partial(
    pl.kernel,
    out_type=jax.ShapeDtypeStruct((V * D,), jnp.float32),
    mesh=_MESH,
    scratch_types=[
        pltpu.VMEM((8, XW), jnp.float32),
        pltpu.VMEM((8, XW), jnp.float32),
        pltpu.VMEM((8, XW), jnp.float32),
        pltpu.VMEM((8, XW), jnp.float32),
        pltpu.VMEM((XW * D,), jnp.float32),
        pltpu.VMEM((XW * D,), jnp.float32),
        pltpu.SemaphoreType.DMA,
        pltpu.SemaphoreType.DMA,
        pltpu.SemaphoreType.DMA,
        pltpu.SemaphoreType.DMA,
    ],
    compiler_params=_PARAMS_TC,
)(_detile_body)


def _sc_body(x_hbm, emb_hbm, un_hbm, z_hbm,
             xraw_v, idx_v, rows_v0, rows_v1, u_v0, u_v1, usum_v, out_v,
             sem0, sem1):
    wid = lax.axis_index("s") * NC + lax.axis_index("c")
    rows_bufs = (rows_v0, rows_v1)
    u_bufs = (u_v0, u_v1)
    sems = (sem0, sem1)

    lane = lax.iota(jnp.int32, L)

    pltpu.sync_copy(x_hbm.at[pl.ds(wid * RPW, RPW)], xraw_v)

    def pbody(k, carry):
        flat = k * L + lane
        vals = plsc.load_gather(xraw_v, [flat // F, flat % F])
        idx_v[k // (XW // L), pl.ds((k % (XW // L)) * L, L)] = vals
        return carry
    lax.fori_loop(0, IPW // L, pbody, 0)

    def issue(c):
        slot = c % 2
        descs = []
        for j in range(GJ):
            row = c * GJ + j
            descs.append(pltpu.async_copy(
                emb_hbm.at[idx_v.at[row]],
                rows_bufs[slot].at[pl.ds(j * XW, XW)], sems[slot]))
            descs.append(pltpu.async_copy(
                un_hbm.at[idx_v.at[row]],
                u_bufs[slot].at[pl.ds(j * XW, XW)], sems[slot]))
        return descs

    descs = issue(0)
    for c in range(NCHUNK):
        nxt = issue(c + 1) if c + 1 < NCHUNK else []
        for dsc in descs:
            dsc.wait()
        descs = nxt
        slot = c % 2
        rows_b = rows_bufs[slot]
        u_b = u_bufs[slot]

        def ubody(g, carry):
            base = g * (L * F)
            acc = jnp.zeros((L,), jnp.float32)
            for f in range(F):
                vals = plsc.load_gather(u_b, [base + lane * F + f])
                acc = acc + vals
            for i in range(L):
                usum_v[g * L + i, :] = jnp.broadcast_to(acc[i], (D,))
            return carry
        lax.fori_loop(0, GROUPS, ubody, 0)

        def rbody(r, carry):
            acc = jnp.zeros((D,), jnp.float32)
            acc2 = jnp.zeros((D,), jnp.float32)
            for f in range(F):
                v = rows_b[r * F + f, :]
                acc = acc + v
                acc2 = acc2 + v * v
            out_v[r, :] = acc * acc - acc2 + usum_v[r, :]
            return carry
        lax.fori_loop(0, C, rbody, 0)

        pltpu.sync_copy(out_v, z_hbm.at[pl.ds(wid * RPW + c * C, C)])


_sc_ffm = functools.partial(
    pl.kernel,
    out_type=jax.ShapeDtypeStruct((B, D), jnp.float32),
    mesh=_MESH,
    scratch_types=[
        pltpu.VMEM((RPW, F), jnp.int32),
        pltpu.VMEM((XROWS_W, XW), jnp.int32),
        pltpu.VMEM((C * F, D), jnp.float32),
        pltpu.VMEM((C * F, D), jnp.float32),
        pltpu.VMEM((C * F,), jnp.float32),
        pltpu.VMEM((C * F,), jnp.float32),
        pltpu.VMEM((C, D), jnp.float32),
        pltpu.VMEM((C, D), jnp.float32),
        pltpu.SemaphoreType.DMA,
        pltpu.SemaphoreType.DMA,
    ],
    compiler_params=_PARAMS_LIN,
)(_sc_body)


def _logsig_body(z_ref, o_ref):
    z = z_ref[...]
    o_ref[...] = jnp.where(z >= 0.0,
                           -jnp.log1p(jnp.exp(-z)),
                           z - jnp.log1p(jnp.exp(z)))


def _logsig(z):
    z2 = z.reshape(B * D // 128, 128)
    out = pl.pallas_call(
        _logsig_body,
        out_shape=jax.ShapeDtypeStruct(z2.shape, jnp.float32),
    )(z2)
    return out.reshape(B, D)


def kernel(X, emb_table, unary_table):
    embt = emb_table.T
    tail = jnp.pad(embt[:, TCF * XW:], ((0, 0), (0, XW - TPART)))
    emb_lin = _detile(embt, tail)
    z = _sc_ffm(X, emb_lin.reshape(V, D), unary_table.reshape(-1))
    return _logsig(z)


# speedup vs baseline: 3.7426x; 3.7426x over previous
import functools

import jax
import jax.numpy as jnp
from jax import lax
from jax.experimental import pallas as pl
from jax.experimental.pallas import tpu as pltpu
from jax.experimental.pallas import tpu_sc as plsc

B = 16384
F = 26
D = 16
V = 1000000
NC, NS, L = 2, 16, 16
NW = NC * NS
RPW = B // NW
IPW = RPW * F
XW = 128
XROWS_W = IPW // XW
C = 64
NCHUNK = RPW // C
GJ = C * F // XW
GROUPS = C // L

TCF = V // XW
TPW = TCF // NW
TEXTRA = TCF - TPW * NW
TPART = V - TCF * XW

_MESH = plsc.VectorSubcoreMesh(core_axis_name="c", subcore_axis_name="s")
_PARAMS_LIN = pltpu.CompilerParams(needs_layout_passes=False,
                                   use_tc_tiling_on_sc=False)
_PARAMS_TC = pltpu.CompilerParams(needs_layout_passes=False,
                                  use_tc_tiling_on_sc=True)


def _detile_body(embt_hbm, tail_hbm, lin_hbm,
                 in00, in01, in02, in03, in10, in11, in12, in13,
                 out0, out1, out2, out3,
                 rsem0, rsem1, rsem2, rsem3,
                 wsem0, wsem1, wsem2, wsem3):
    wid = lax.axis_index("s") * NC + lax.axis_index("c")
    base = wid * TPW
    lane = lax.iota(jnp.int32, L)
    ins = ((in00, in01, in02, in03), (in10, in11, in12, in13))
    outs = (out0, out1, out2, out3)
    rsems = (rsem0, rsem1, rsem2, rsem3)
    wsems = (wsem0, wsem1, wsem2, wsem3)

    def read(tc, slot):
        col = pl.multiple_of(tc * XW, XW)
        pltpu.async_copy(embt_hbm.at[pl.ds(0, 8), pl.ds(col, XW)],
                         ins[0][slot], rsems[slot])
        pltpu.async_copy(embt_hbm.at[pl.ds(8, 8), pl.ds(col, XW)],
                         ins[1][slot], rsems[slot])

    def drain_read(slot):
        for h in range(2):
            pltpu.make_async_copy(
                embt_hbm.at[pl.ds(0, 8), pl.ds(0, XW)],
                ins[h][slot], rsems[slot]).wait()

    def drain_write(slot):
        pltpu.make_async_copy(
            outs[slot], lin_hbm.at[pl.ds(0, D * XW)], wsems[slot]).wait()

    scat = lane * D

    def transpose(slot, width):
        ob = outs[slot]

        def tbody(k16, carry):
            col = k16 * L
            for d in range(D):
                vals = ins[d // 8][slot][d % 8, pl.ds(col, L)]
                plsc.store_scatter(ob, [scat + (k16 * (L * D) + d)], vals)
            return carry
        lax.fori_loop(0, width // L, tbody, 0)

    NBUF = 4
    for b in range(NBUF):
        read(base + b, b)

    def body(it, carry):
        for b in range(NBUF):
            blk = it * NBUF + b
            drain_read(b)

            @pl.when(blk >= NBUF)
            def _():
                drain_write(b)

            transpose(b, XW)
            off = pl.multiple_of((base + blk) * (XW * D), XW * D)
            pltpu.async_copy(outs[b], lin_hbm.at[pl.ds(off, XW * D)],
                             wsems[b])

            @pl.when(blk + NBUF < TPW)
            def _():
                read(base + blk + NBUF, b)
        return carry
    lax.fori_loop(0, TPW // NBUF, body, 0)
    for b in range(NBUF):
        drain_write(b)

    @pl.when(wid < TEXTRA)
    def _():
        tc = TCF - TEXTRA + wid
        col = pl.multiple_of(tc * XW, XW)
        pltpu.sync_copy(embt_hbm.at[pl.ds(0, 8), pl.ds(col, XW)], in00)
        pltpu.sync_copy(embt_hbm.at[pl.ds(8, 8), pl.ds(col, XW)], in10)
        transpose(0, XW)
        off = pl.multiple_of(tc * (XW * D), XW * D)
        pltpu.sync_copy(out0, lin_hbm.at[pl.ds(off, XW * D)])

    @pl.when(wid == TEXTRA)
    def _():
        pltpu.sync_copy(tail_hbm.at[pl.ds(0, 8), pl.ds(0, XW)], in00)
        pltpu.sync_copy(tail_hbm.at[pl.ds(8, 8), pl.ds(0, XW)], in10)
        transpose(0, TPART)
        pltpu.sync_copy(out0.at[pl.ds(0, TPART * D)],
                        lin_hbm.at[pl.ds(TCF * XW * D, TPART * D)])


_detile = functools.partial(
    pl.kernel,
    out_type=jax.ShapeDtypeStruct((V * D,), jnp.float32),
    mesh=_MESH,
    scratch_types=(
        [pltpu.VMEM((8, XW), jnp.float32)] * 8
        + [pltpu.VMEM((XW * D,), jnp.float32)] * 4
        + [pltpu.SemaphoreType.DMA] * 8
    ),
    compiler_params=_PARAMS_TC,
)(_detile_body)


def _sc_body(x_hbm, emb_hbm, un_hbm, z_hbm,
             xraw_v, idx_v, rows_v0, rows_v1, u_v0, u_v1, usum_v, out_v,
             sem0, sem1):
    wid = lax.axis_index("s") * NC + lax.axis_index("c")
    rows_bufs = (rows_v0, rows_v1)
    u_bufs = (u_v0, u_v1)
    sems = (sem0, sem1)

    lane = lax.iota(jnp.int32, L)

    pltpu.sync_copy(x_hbm.at[pl.ds(wid * RPW, RPW)], xraw_v)

    def pbody(k, carry):
        flat = k * L + lane
        vals = plsc.load_gather(xraw_v, [flat // F, flat % F])
        idx_v[k // (XW // L), pl.ds((k % (XW // L)) * L, L)] = vals
        return carry
    lax.fori_loop(0, IPW // L, pbody, 0)

    def issue(c):
        slot = c % 2
        descs = []
        for j in range(GJ):
            row = c * GJ + j
            descs.append(pltpu.async_copy(
                emb_hbm.at[idx_v.at[row]],
                rows_bufs[slot].at[pl.ds(j * XW, XW)], sems[slot]))
            descs.append(pltpu.async_copy(
                un_hbm.at[idx_v.at[row]],
                u_bufs[slot].at[pl.ds(j * XW, XW)], sems[slot]))
        return descs

    descs = issue(0)
    for c in range(NCHUNK):
        nxt = issue(c + 1) if c + 1 < NCHUNK else []
        for dsc in descs:
            dsc.wait()
        descs = nxt
        slot = c % 2
        rows_b = rows_bufs[slot]
        u_b = u_bufs[slot]

        def ubody(g, carry):
            base = g * (L * F)
            acc = jnp.zeros((L,), jnp.float32)
            for f in range(F):
                vals = plsc.load_gather(u_b, [base + lane * F + f])
                acc = acc + vals
            for i in range(L):
                usum_v[g * L + i, :] = jnp.broadcast_to(acc[i], (D,))
            return carry
        lax.fori_loop(0, GROUPS, ubody, 0)

        def rbody(r, carry):
            acc = jnp.zeros((D,), jnp.float32)
            acc2 = jnp.zeros((D,), jnp.float32)
            for f in range(F):
                v = rows_b[r * F + f, :]
                acc = acc + v
                acc2 = acc2 + v * v
            out_v[r, :] = acc * acc - acc2 + usum_v[r, :]
            return carry
        lax.fori_loop(0, C, rbody, 0)

        pltpu.sync_copy(out_v, z_hbm.at[pl.ds(wid * RPW + c * C, C)])


_sc_ffm = functools.partial(
    pl.kernel,
    out_type=jax.ShapeDtypeStruct((B, D), jnp.float32),
    mesh=_MESH,
    scratch_types=[
        pltpu.VMEM((RPW, F), jnp.int32),
        pltpu.VMEM((XROWS_W, XW), jnp.int32),
        pltpu.VMEM((C * F, D), jnp.float32),
        pltpu.VMEM((C * F, D), jnp.float32),
        pltpu.VMEM((C * F,), jnp.float32),
        pltpu.VMEM((C * F,), jnp.float32),
        pltpu.VMEM((C, D), jnp.float32),
        pltpu.VMEM((C, D), jnp.float32),
        pltpu.SemaphoreType.DMA,
        pltpu.SemaphoreType.DMA,
    ],
    compiler_params=_PARAMS_LIN,
)(_sc_body)


def _logsig_body(z_ref, o_ref):
    z = z_ref[...]
    o_ref[...] = jnp.where(z >= 0.0,
                           -jnp.log1p(jnp.exp(-z)),
                           z - jnp.log1p(jnp.exp(z)))


def _logsig(z):
    z2 = z.reshape(B * D // 128, 128)
    out = pl.pallas_call(
        _logsig_body,
        out_shape=jax.ShapeDtypeStruct(z2.shape, jnp.float32),
    )(z2)
    return out.reshape(B, D)


def kernel(X, emb_table, unary_table):
    embt = emb_table.T
    tail = jnp.pad(embt[:, TCF * XW:], ((0, 0), (0, XW - TPART)))
    emb_lin = _detile(embt, tail)
    z = _sc_ffm(X, emb_lin.reshape(V, D), unary_table.reshape(-1))
    return _logsig(z)


# speedup vs baseline: 3.7547x; 1.0032x over previous
import functools

import jax
import jax.numpy as jnp
from jax import lax
from jax.experimental import pallas as pl
from jax.experimental.pallas import tpu as pltpu
from jax.experimental.pallas import tpu_sc as plsc

B = 16384
F = 26
D = 16
V = 1000000
NC, NS, L = 2, 16, 16
NW = NC * NS
RPW = B // NW
IPW = RPW * F
XW = 128
XROWS_W = IPW // XW
C = 64
NCHUNK = RPW // C
GJ = C * F // XW
GROUPS = C // L

TCW = 512
TPW = 61
TFULL = TPW * NW
TPART = V - (TFULL + 1) * TCW

_MESH = plsc.VectorSubcoreMesh(core_axis_name="c", subcore_axis_name="s")
_PARAMS_LIN = pltpu.CompilerParams(needs_layout_passes=False,
                                   use_tc_tiling_on_sc=False)
_PARAMS_TC = pltpu.CompilerParams(needs_layout_passes=False,
                                  use_tc_tiling_on_sc=True)


def _detile_body(embt_hbm, tail_hbm, lin_hbm,
                 in00, in01, in02, in03, in10, in11, in12, in13,
                 out0, out1, out2, out3,
                 rsem0, rsem1, rsem2, rsem3,
                 wsem0, wsem1, wsem2, wsem3):
    wid = lax.axis_index("s") * NC + lax.axis_index("c")
    base = wid * TPW
    lane = lax.iota(jnp.int32, L)
    ins = ((in00, in01, in02, in03), (in10, in11, in12, in13))
    outs = (out0, out1, out2, out3)
    rsems = (rsem0, rsem1, rsem2, rsem3)
    wsems = (wsem0, wsem1, wsem2, wsem3)

    def read(tc, slot):
        col = pl.multiple_of(tc * TCW, TCW)
        pltpu.async_copy(embt_hbm.at[pl.ds(0, 8), pl.ds(col, TCW)],
                         ins[0][slot], rsems[slot])
        pltpu.async_copy(embt_hbm.at[pl.ds(8, 8), pl.ds(col, TCW)],
                         ins[1][slot], rsems[slot])

    def drain_read(slot):
        for h in range(2):
            pltpu.make_async_copy(
                embt_hbm.at[pl.ds(0, 8), pl.ds(0, TCW)],
                ins[h][slot], rsems[slot]).wait()

    def drain_write(slot):
        pltpu.make_async_copy(
            outs[slot], lin_hbm.at[pl.ds(0, D * TCW)], wsems[slot]).wait()

    scat = lane * D

    def transpose(slot, width):
        ob = outs[slot]

        def tbody(k16, carry):
            col = k16 * L
            for d in range(D):
                vals = ins[d // 8][slot][d % 8, pl.ds(col, L)]
                plsc.store_scatter(ob, [scat + (k16 * (L * D) + d)], vals)
            return carry
        lax.fori_loop(0, width // L, tbody, 0)

    NBUF = 4
    for b in range(NBUF):
        read(base + b, b)

    def body(it, carry):
        for b in range(NBUF):
            blk = it * NBUF + b
            drain_read(b)

            @pl.when(blk >= NBUF)
            def _():
                drain_write(b)

            transpose(b, TCW)
            off = pl.multiple_of((base + blk) * (TCW * D), TCW * D)
            pltpu.async_copy(outs[b], lin_hbm.at[pl.ds(off, TCW * D)],
                             wsems[b])

            @pl.when(blk + NBUF < TPW)
            def _():
                read(base + blk + NBUF, b)
        return carry
    lax.fori_loop(0, (TPW - 1) // NBUF, body, 0)

    drain_read(0)
    drain_write(0)
    transpose(0, TCW)
    off0 = pl.multiple_of((base + TPW - 1) * (TCW * D), TCW * D)
    pltpu.async_copy(outs[0], lin_hbm.at[pl.ds(off0, TCW * D)], wsems[0])
    for b in range(1, NBUF):
        drain_write(b)

    @pl.when(wid == 0)
    def _():
        col = TFULL * TCW
        pltpu.sync_copy(embt_hbm.at[pl.ds(0, 8), pl.ds(col, TCW)], in01)
        pltpu.sync_copy(embt_hbm.at[pl.ds(8, 8), pl.ds(col, TCW)], in11)
        transpose(1, TCW)
        pltpu.sync_copy(out1, lin_hbm.at[pl.ds(col * D, TCW * D)])

    @pl.when(wid == 1)
    def _():
        pltpu.sync_copy(tail_hbm.at[pl.ds(0, 8), pl.ds(0, TCW)], in01)
        pltpu.sync_copy(tail_hbm.at[pl.ds(8, 8), pl.ds(0, TCW)], in11)
        transpose(1, TPART)
        pltpu.sync_copy(out1.at[pl.ds(0, TPART * D)],
                        lin_hbm.at[pl.ds((TFULL + 1) * TCW * D, TPART * D)])

    drain_write(0)


_detile = functools.partial(
    pl.kernel,
    out_type=jax.ShapeDtypeStruct((V * D,), jnp.float32),
    mesh=_MESH,
    scratch_types=(
        [pltpu.VMEM((8, TCW), jnp.float32)] * 8
        + [pltpu.VMEM((TCW * D,), jnp.float32)] * 4
        + [pltpu.SemaphoreType.DMA] * 8
    ),
    compiler_params=_PARAMS_TC,
)(_detile_body)


def _sc_body(x_hbm, emb_hbm, un_hbm, z_hbm,
             xraw_v, idx_v, rows_v0, rows_v1, u_v0, u_v1, usum_v, out_v,
             sem0, sem1):
    wid = lax.axis_index("s") * NC + lax.axis_index("c")
    rows_bufs = (rows_v0, rows_v1)
    u_bufs = (u_v0, u_v1)
    sems = (sem0, sem1)

    lane = lax.iota(jnp.int32, L)

    pltpu.sync_copy(x_hbm.at[pl.ds(wid * RPW, RPW)], xraw_v)

    def pbody(k, carry):
        flat = k * L + lane
        vals = plsc.load_gather(xraw_v, [flat // F, flat % F])
        idx_v[k // (XW // L), pl.ds((k % (XW // L)) * L, L)] = vals
        return carry
    lax.fori_loop(0, IPW // L, pbody, 0)

    def issue(c):
        slot = c % 2
        descs = []
        for j in range(GJ):
            row = c * GJ + j
            descs.append(pltpu.async_copy(
                emb_hbm.at[idx_v.at[row]],
                rows_bufs[slot].at[pl.ds(j * XW, XW)], sems[slot]))
            descs.append(pltpu.async_copy(
                un_hbm.at[idx_v.at[row]],
                u_bufs[slot].at[pl.ds(j * XW, XW)], sems[slot]))
        return descs

    descs = issue(0)
    for c in range(NCHUNK):
        nxt = issue(c + 1) if c + 1 < NCHUNK else []
        for dsc in descs:
            dsc.wait()
        descs = nxt
        slot = c % 2
        rows_b = rows_bufs[slot]
        u_b = u_bufs[slot]

        def ubody(g, carry):
            base = g * (L * F)
            acc = jnp.zeros((L,), jnp.float32)
            for f in range(F):
                vals = plsc.load_gather(u_b, [base + lane * F + f])
                acc = acc + vals
            for i in range(L):
                usum_v[g * L + i, :] = jnp.broadcast_to(acc[i], (D,))
            return carry
        lax.fori_loop(0, GROUPS, ubody, 0)

        def rbody(r, carry):
            acc = jnp.zeros((D,), jnp.float32)
            acc2 = jnp.zeros((D,), jnp.float32)
            for f in range(F):
                v = rows_b[r * F + f, :]
                acc = acc + v
                acc2 = acc2 + v * v
            out_v[r, :] = acc * acc - acc2 + usum_v[r, :]
            return carry
        lax.fori_loop(0, C, rbody, 0)

        pltpu.sync_copy(out_v, z_hbm.at[pl.ds(wid * RPW + c * C, C)])


_sc_ffm = functools.partial(
    pl.kernel,
    out_type=jax.ShapeDtypeStruct((B, D), jnp.float32),
    mesh=_MESH,
    scratch_types=[
        pltpu.VMEM((RPW, F), jnp.int32),
        pltpu.VMEM((XROWS_W, XW), jnp.int32),
        pltpu.VMEM((C * F, D), jnp.float32),
        pltpu.VMEM((C * F, D), jnp.float32),
        pltpu.VMEM((C * F,), jnp.float32),
        pltpu.VMEM((C * F,), jnp.float32),
        pltpu.VMEM((C, D), jnp.float32),
        pltpu.VMEM((C, D), jnp.float32),
        pltpu.SemaphoreType.DMA,
        pltpu.SemaphoreType.DMA,
    ],
    compiler_params=_PARAMS_LIN,
)(_sc_body)


def _logsig_body(z_ref, o_ref):
    z = z_ref[...]
    o_ref[...] = jnp.where(z >= 0.0,
                           -jnp.log1p(jnp.exp(-z)),
                           z - jnp.log1p(jnp.exp(z)))


def _logsig(z):
    z2 = z.reshape(B * D // 128, 128)
    out = pl.pallas_call(
        _logsig_body,
        out_shape=jax.ShapeDtypeStruct(z2.shape, jnp.float32),
    )(z2)
    return out.reshape(B, D)


def kernel(X, emb_table, unary_table):
    embt = emb_table.T
    tail = jnp.pad(embt[:, (TFULL + 1) * TCW:], ((0, 0), (0, TCW - TPART)))
    emb_lin = _detile(embt, tail)
    z = _sc_ffm(X, emb_lin.reshape(V, D), unary_table.reshape(-1))
    return _logsig(z)
